# baseline (speedup 1.0000x reference)
import functools

import jax
import jax.numpy as jnp
from jax import lax
from jax.experimental import pallas as pl
from jax.experimental.pallas import tpu as pltpu
from jax.experimental.pallas import tpu_sc as plsc

B, H, T, D = 2, 16, 4096, 128
C = 64
W = 64
COMMITMENT = 0.0001

NC, NS = 2, 16
NW = NC * NS
CG = C // 16
NGRP = B * H * CG
GRP_PER_W = NGRP // NW
KST = 17
CHUNK = 512
NCHUNK = T // CHUNK

_MININT = -(1 << 31)
_M7F = (1 << 31) - 1


def _tc_body(x_ref, m_ref, d_ref, l_ref):
    xb = x_ref[0, 0]
    mb = m_ref[0]
    dt = lax.dot_general(mb, xb, (((1,), (1,)), ((), ())),
                         preferred_element_type=jnp.float32)
    d_ref[0, 0] = dt.reshape(CG, 16, T)
    colmax = jnp.max(dt, axis=0, keepdims=True)
    cio = lax.broadcasted_iota(jnp.int32, (C, T), 0)
    amax = jnp.min(jnp.where(dt == colmax, cio, C), axis=0,
                   keepdims=True)
    mnorm = jnp.sum(mb * mb, axis=1, keepdims=True)
    nsel = jnp.sum(jnp.where(cio == amax, mnorm, 0.0), axis=0)
    part = (jnp.sum(xb * xb) - 2.0 * jnp.sum(colmax) + jnp.sum(nsel))
    l_ref[...] = part.reshape(1, 1, 1)


def _make_tc_call(boff):
    return pl.pallas_call(
        _tc_body,
        grid=(H,),
        in_specs=[
            pl.BlockSpec((1, 1, T, D), lambda i, _b=boff: (_b, i, 0, 0)),
            pl.BlockSpec((1, C, D), lambda i: (i, 0, 0)),
        ],
        out_specs=[
            pl.BlockSpec((1, 1, CG, 16, T), lambda i: (0, i, 0, 0, 0)),
            pl.BlockSpec((1, 1, 1), lambda i: (i, 0, 0)),
        ],
        out_shape=[
            jax.ShapeDtypeStruct((1, H, CG, 16, T), jnp.float32),
            jax.ShapeDtypeStruct((H, 1, 1), jnp.float32),
        ],
        compiler_params=pltpu.CompilerParams(
            dimension_semantics=("arbitrary",)),
    )


def _sc_topk(gpw, d_hbm, out_hbm, stage_a, stage_b, kt_v, hist_v, out_v,
             sem_a, sem_b):
    wid = lax.axis_index("s") * NC + lax.axis_index("c")
    lane = lax.iota(jnp.int32, 16)
    lane_kst = lane * KST
    ones16 = jnp.ones((16,), jnp.int32)
    zeros16 = jnp.zeros((16,), jnp.int32)

    def zero_body(i, c):
        for j in range(4):
            hist_v[i * 4 + j] = zeros16
        return c
    lax.fori_loop(0, 64, zero_body, 0)

    def group_body(gi, _carry):
        g = wid * gpw + gi
        b = g // (H * CG)
        h = (g // CG) % H
        cg = g % CG

        stages = (stage_a, stage_b)
        sems = (sem_a, sem_b)
        copies = [None] * NCHUNK
        copies[0] = pltpu.async_copy(
            d_hbm.at[b, h, cg, :, pl.ds(0, CHUNK)], stages[0], sems[0])
        for ci in range(NCHUNK):
            copies[ci].wait()
            if ci + 1 < NCHUNK:
                copies[ci + 1] = pltpu.async_copy(
                    d_hbm.at[b, h, cg, :, pl.ds((ci + 1) * CHUNK, CHUNK)],
                    stages[(ci + 1) % 2], sems[(ci + 1) % 2])
            stage_v = stages[ci % 2]

            @plsc.parallel_loop(0, 16 * (CHUNK // 16), unroll=16)
            def _imp(n, _ci=ci, _sv=stage_v):
                j = n // (CHUNK // 16)
                i = n % (CHUNK // 16)
                v = _sv[j, pl.ds(i * 16, 16)] + 0.0
                u = plsc.bitcast(v, jnp.int32)
                ks = u ^ ((u >> 31) & _M7F)
                ku = ks ^ _MININT
                base = KST * (_ci * CHUNK + i * 16) + j
                addr = jnp.full((16,), base, jnp.int32) + KST * lane
                plsc.store_scatter(kt_v, [addr], ku)

        pu = zeros16
        rem = jnp.full((16,), W, jnp.int32)

        for p in range(4):
            s = 24 - 8 * p

            if p == 0:
                @plsc.parallel_loop(0, T, unroll=16)
                def _hist0(t):
                    addr = jnp.full((16,), KST * t, jnp.int32) + lane
                    ku = plsc.load_gather(kt_v, [addr])
                    digit = (ku >> 24) & 255
                    plsc.addupdate_scatter(hist_v, [digit, lane], ones16)
            else:
                himask = -(1 << (s + 8))

                def _histp(t, _s=s, _hm=himask):
                    addr = jnp.full((16,), KST * t, jnp.int32) + lane
                    ku = plsc.load_gather(kt_v, [addr])
                    digit = (ku >> _s) & 255
                    cand = (ku & _hm) == pu
                    plsc.addupdate_scatter(hist_v, [digit, lane], ones16,
                                           mask=cand)
                plsc.parallel_loop(0, T, unroll=16)(_histp)

            def scan_body(i, st):
                acc, found, dsel, rem2 = st
                for j in range(4):
                    d = 255 - (i * 4 + j)
                    hrow = hist_v[d]
                    hist_v[d] = zeros16
                    acc2 = acc + hrow
                    new = (acc2 >= rem) & (found == 0)
                    dsel = jnp.where(new, d, dsel)
                    rem2 = jnp.where(new, rem - acc, rem2)
                    found = jnp.where(new, ones16, found)
                    acc = acc2
                return (acc, found, dsel, rem2)

            _, _, dsel, rem = lax.fori_loop(
                0, 64, scan_body, (zeros16, zeros16, zeros16, rem))
            pu = pu | (dsel << s)

        ts = pu ^ _MININT

        def sel_body(t, st):
            cnt, tie = st
            addr = jnp.full((16,), KST * t, jnp.int32) + lane
            ku = plsc.load_gather(kt_v, [addr])
            ks = ku ^ _MININT
            gt = ks > ts
            eq = ku == pu
            sel = gt | (eq & (tie < rem))
            tv = jnp.full((16,), t, jnp.int32)
            plsc.store_scatter(out_v, [lane, cnt], tv, mask=sel)
            return (cnt + sel.astype(jnp.int32), tie + eq.astype(jnp.int32))

        plsc.parallel_loop(0, T, unroll=16,
                           carry=(zeros16, zeros16))(sel_body)

        pltpu.sync_copy(out_v, out_hbm.at[b, h, pl.ds(cg * 16, 16), :])
        return _carry

    lax.fori_loop(0, gpw, group_body, 0)


def _make_sc_call(Bs):
    gpw = Bs * H * CG // NW
    return functools.partial(
        pl.kernel,
        out_type=jax.ShapeDtypeStruct((Bs, H, C, W), jnp.int32),
        mesh=plsc.VectorSubcoreMesh(core_axis_name="c", subcore_axis_name="s",
                                    num_cores=NC, num_subcores=NS),
        scratch_types=[
            pltpu.VMEM((16, CHUNK), jnp.float32),
            pltpu.VMEM((16, CHUNK), jnp.float32),
            pltpu.VMEM((T * KST,), jnp.int32),
            pltpu.VMEM((256, 16), jnp.int32),
            pltpu.VMEM((16, W), jnp.int32),
            pltpu.SemaphoreType.DMA,
            pltpu.SemaphoreType.DMA,
        ],
        compiler_params=pltpu.CompilerParams(use_tc_tiling_on_sc=True,
                                             needs_layout_passes=False),
    )(functools.partial(_sc_topk, gpw))


_tc_halves = [_make_tc_call(i) for i in range(B)]
_sc_half = _make_sc_call(1)


def kernel(x, window_size, means):
    idxs, losses = [], []
    for i in range(B):
        d_i, l_i = _tc_halves[i](x, means)
        idxs.append(_sc_half(d_i))
        losses.append(jnp.sum(l_i))
    idx = jnp.concatenate(idxs, axis=0)
    indices = idx.reshape(B, H, C * W) + (window_size - W)
    loss = (losses[0] + losses[1]) * (COMMITMENT / (B * H * T * D))
    return (indices, loss)


# speedup vs baseline: 1.1946x; 1.1946x over previous
import functools

import jax
import jax.numpy as jnp
from jax import lax
from jax.experimental import pallas as pl
from jax.experimental.pallas import tpu as pltpu
from jax.experimental.pallas import tpu_sc as plsc

B, H, T, D = 2, 16, 4096, 128
C = 64
W = 64
COMMITMENT = 0.0001

NC, NS = 2, 16
NW = NC * NS
CG = C // 16
NGRP = B * H * CG
GRP_PER_W = NGRP // NW
KST = 17
CHUNK = 512
NCHUNK = T // CHUNK

_MININT = -(1 << 31)
_M7F = (1 << 31) - 1


def _tc_body(x_ref, m_ref, d_ref, l_ref):
    xb = x_ref[0, 0]
    mb = m_ref[0]
    dt = lax.dot_general(mb, xb, (((1,), (1,)), ((), ())),
                         preferred_element_type=jnp.float32)
    d_ref[0, 0] = dt.reshape(CG, 16, T)
    colmax = jnp.max(dt, axis=0, keepdims=True)
    cio = lax.broadcasted_iota(jnp.int32, (C, T), 0)
    amax = jnp.min(jnp.where(dt == colmax, cio, C), axis=0,
                   keepdims=True)
    mnorm = jnp.sum(mb * mb, axis=1, keepdims=True)
    nsel = jnp.sum(jnp.where(cio == amax, mnorm, 0.0), axis=0)
    part = (jnp.sum(xb * xb) - 2.0 * jnp.sum(colmax) + jnp.sum(nsel))
    l_ref[...] = part.reshape(1, 1, 1)


def _make_tc_call(boff):
    return pl.pallas_call(
        _tc_body,
        grid=(H,),
        in_specs=[
            pl.BlockSpec((1, 1, T, D), lambda i, _b=boff: (_b, i, 0, 0)),
            pl.BlockSpec((1, C, D), lambda i: (i, 0, 0)),
        ],
        out_specs=[
            pl.BlockSpec((1, 1, CG, 16, T), lambda i: (0, i, 0, 0, 0)),
            pl.BlockSpec((1, 1, 1), lambda i: (i, 0, 0)),
        ],
        out_shape=[
            jax.ShapeDtypeStruct((1, H, CG, 16, T), jnp.float32),
            jax.ShapeDtypeStruct((H, 1, 1), jnp.float32),
        ],
        compiler_params=pltpu.CompilerParams(
            dimension_semantics=("arbitrary",)),
    )


def _sc_topk(gpw, d_hbm, out_hbm, stage_a, stage_b, kt_v, hist_v, out_v,
             sem_a, sem_b):
    wid = lax.axis_index("s") * NC + lax.axis_index("c")
    lane = lax.iota(jnp.int32, 16)
    lane_kst = lane * KST
    ones16 = jnp.ones((16,), jnp.int32)
    zeros16 = jnp.zeros((16,), jnp.int32)

    def zero_body(i, c):
        for j in range(4):
            hist_v[i * 4 + j] = zeros16
        return c
    lax.fori_loop(0, 64, zero_body, 0)

    def group_body(gi, _carry):
        g = wid * gpw + gi
        b = g // (H * CG)
        h = (g // CG) % H
        cg = g % CG

        stages = (stage_a, stage_b)
        sems = (sem_a, sem_b)
        copies = [None] * NCHUNK
        copies[0] = pltpu.async_copy(
            d_hbm.at[b, h, cg, :, pl.ds(0, CHUNK)], stages[0], sems[0])
        for ci in range(NCHUNK):
            copies[ci].wait()
            if ci + 1 < NCHUNK:
                copies[ci + 1] = pltpu.async_copy(
                    d_hbm.at[b, h, cg, :, pl.ds((ci + 1) * CHUNK, CHUNK)],
                    stages[(ci + 1) % 2], sems[(ci + 1) % 2])
            stage_v = stages[ci % 2]

            @plsc.parallel_loop(0, 16 * (CHUNK // 16), unroll=8)
            def _imp(n, _ci=ci, _sv=stage_v):
                j = n // (CHUNK // 16)
                i = n % (CHUNK // 16)
                v = _sv[j, pl.ds(i * 16, 16)] + 0.0
                u = plsc.bitcast(v, jnp.int32)
                ks = u ^ ((u >> 31) & _M7F)
                ku = ks ^ _MININT
                base = KST * (_ci * CHUNK + i * 16) + j
                addr = jnp.full((16,), base, jnp.int32) + KST * lane
                plsc.store_scatter(kt_v, [addr], ku)

        pu = zeros16
        rem = jnp.full((16,), W, jnp.int32)
        done = zeros16

        def radix_pass(p, pu, rem, done):
            s = 24 - 8 * p

            if p == 0:
                @plsc.parallel_loop(0, T, unroll=8)
                def _hist0(t):
                    addr = jnp.full((16,), KST * t, jnp.int32) + lane
                    ku = plsc.load_gather(kt_v, [addr])
                    digit = (ku >> 24) & 255
                    plsc.addupdate_scatter(hist_v, [digit, lane], ones16)
            else:
                himask = -(1 << (s + 8))

                def _histp(t, _s=s, _hm=himask):
                    addr = jnp.full((16,), KST * t, jnp.int32) + lane
                    ku = plsc.load_gather(kt_v, [addr])
                    digit = (ku >> _s) & 255
                    cand = (ku & _hm) == pu
                    plsc.addupdate_scatter(hist_v, [digit, lane], ones16,
                                           mask=cand)
                plsc.parallel_loop(0, T, unroll=8)(_histp)

            def scan_body(i, st):
                acc, found, dsel, rem2, hsel = st
                for j in range(4):
                    d = 255 - (i * 4 + j)
                    hrow = hist_v[d]
                    hist_v[d] = zeros16
                    acc2 = acc + hrow
                    new = (acc2 >= rem) & (found == 0)
                    dsel = jnp.where(new, d, dsel)
                    rem2 = jnp.where(new, rem - acc, rem2)
                    hsel = jnp.where(new, hrow, hsel)
                    found = jnp.where(new, ones16, found)
                    acc = acc2
                return (acc, found, dsel, rem2, hsel)

            _, _, dsel, rem2, hsel = lax.fori_loop(
                0, 64, scan_body, (zeros16, zeros16, zeros16, rem, zeros16))
            prefix = pu | (dsel << s)
            newly = (done == 0) & (rem2 == hsel) & (prefix != 0)
            live = done == 0
            pu = jnp.where(live, jnp.where(newly, prefix - 1, prefix), pu)
            rem = jnp.where(newly, zeros16, jnp.where(live, rem2, rem))
            done = done | newly.astype(jnp.int32)
            return pu, rem, done

        pu, rem, done = radix_pass(0, pu, rem, done)
        for p in range(1, 4):
            pu, rem, done = lax.cond(
                jnp.sum(done) == 16,
                lambda pu=pu, rem=rem, done=done: (pu, rem, done),
                functools.partial(radix_pass, p, pu, rem, done))

        ts = pu ^ _MININT

        def sel_body(t, st):
            cnt, tie = st
            addr = jnp.full((16,), KST * t, jnp.int32) + lane
            ku = plsc.load_gather(kt_v, [addr])
            ks = ku ^ _MININT
            gt = ks > ts
            eq = ku == pu
            sel = gt | (eq & (tie < rem))
            tv = jnp.full((16,), t, jnp.int32)
            plsc.store_scatter(out_v, [lane, cnt], tv, mask=sel)
            return (cnt + sel.astype(jnp.int32), tie + eq.astype(jnp.int32))

        plsc.parallel_loop(0, T, unroll=8,
                           carry=(zeros16, zeros16))(sel_body)

        pltpu.sync_copy(out_v, out_hbm.at[b, h, pl.ds(cg * 16, 16), :])
        return _carry

    lax.fori_loop(0, gpw, group_body, 0)


def _make_sc_call(Bs):
    gpw = Bs * H * CG // NW
    return functools.partial(
        pl.kernel,
        out_type=jax.ShapeDtypeStruct((Bs, H, C, W), jnp.int32),
        mesh=plsc.VectorSubcoreMesh(core_axis_name="c", subcore_axis_name="s",
                                    num_cores=NC, num_subcores=NS),
        scratch_types=[
            pltpu.VMEM((16, CHUNK), jnp.float32),
            pltpu.VMEM((16, CHUNK), jnp.float32),
            pltpu.VMEM((T * KST,), jnp.int32),
            pltpu.VMEM((256, 16), jnp.int32),
            pltpu.VMEM((16, W), jnp.int32),
            pltpu.SemaphoreType.DMA,
            pltpu.SemaphoreType.DMA,
        ],
        compiler_params=pltpu.CompilerParams(use_tc_tiling_on_sc=True,
                                             needs_layout_passes=False),
    )(functools.partial(_sc_topk, gpw))


_tc_halves = [_make_tc_call(i) for i in range(B)]
_sc_half = _make_sc_call(1)


def kernel(x, window_size, means):
    idxs, losses = [], []
    for i in range(B):
        d_i, l_i = _tc_halves[i](x, means)
        idxs.append(_sc_half(d_i))
        losses.append(jnp.sum(l_i))
    idx = jnp.concatenate(idxs, axis=0)
    indices = idx.reshape(B, H, C * W) + (window_size - W)
    loss = (losses[0] + losses[1]) * (COMMITMENT / (B * H * T * D))
    return (indices, loss)
